# baseline (speedup 1.0000x reference)
import functools

import jax
import jax.numpy as jnp
from jax import lax
from jax.experimental import pallas as pl
from jax.experimental.pallas import tpu as pltpu
from jax.experimental.pallas import tpu_sc as plsc

_CHUNK = 128


def _sc_gather(table, idx_all, NW, T, n_chunks):
    V, EMB = table.shape
    b_per_w = n_chunks * _CHUNK
    B = NW * b_per_w
    mesh = plsc.VectorSubcoreMesh(core_axis_name="c", subcore_axis_name="s")
    info = plsc.get_sparse_core_info()
    NC = info.num_cores

    @functools.partial(
        pl.kernel,
        mesh=mesh,
        compiler_params=pltpu.CompilerParams(use_tc_tiling_on_sc=False),
        out_type=jax.ShapeDtypeStruct((T, B, EMB), jnp.float32),
        scratch_types=[
            pltpu.VMEM((T * n_chunks, _CHUNK), jnp.int32),
            pltpu.VMEM((2, _CHUNK, EMB), jnp.float32),
            pltpu.SemaphoreType.DMA,
            pltpu.SemaphoreType.DMA,
        ],
    )
    def k(table_hbm, idx_hbm, out_hbm, idx_v, rows_v, gsem, osem):
        wid = lax.axis_index("s") * NC + lax.axis_index("c")
        pltpu.sync_copy(idx_hbm.at[wid], idx_v)
        nj = T * n_chunks

        def out_slice(j):
            t, c = divmod(j, n_chunks)
            base = wid * b_per_w + c * _CHUNK
            return out_hbm.at[t, pl.ds(base, _CHUNK), :]

        g_cp = [None, None]
        out_cp = [None, None]
        for j in range(nj):
            buf = j % 2
            if out_cp[buf] is not None:
                out_cp[buf].wait()
            g_cp[buf] = pltpu.async_copy(
                table_hbm.at[idx_v.at[j]], rows_v.at[buf], gsem)
            if j > 0:
                ob = 1 - buf
                g_cp[ob].wait()
                out_cp[ob] = pltpu.async_copy(
                    rows_v.at[ob], out_slice(j - 1), osem)
        last = (nj - 1) % 2
        g_cp[last].wait()
        out_cp[last] = pltpu.async_copy(
            rows_v.at[last], out_slice(nj - 1), osem)
        out_cp[1 - last].wait()
        out_cp[last].wait()

    return k(table, idx_all)


def _tc_loss(gathered, center, T, B, EMB, bb):
    nblk = B // bb

    def body(g_ref, c_ref, o_ref):
        i = pl.program_id(0)

        @pl.when(i == 0)
        def _():
            o_ref[0, 0] = 0.0

        c = c_ref[...]
        total = jnp.float32(0.0)
        for t in range(T):
            s = jnp.sum(g_ref[t] * c, axis=1, keepdims=True)
            x = -s if t == 0 else s
            sp = jnp.maximum(x, 0.0) + jnp.log1p(jnp.exp(-jnp.abs(x)))
            total = total + jnp.sum(sp)
        o_ref[0, 0] += total * (1.0 / B)

    out = pl.pallas_call(
        body,
        grid=(nblk,),
        in_specs=[
            pl.BlockSpec((T, bb, EMB), lambda i: (0, i, 0)),
            pl.BlockSpec((bb, EMB), lambda i: (i, 0)),
        ],
        out_specs=pl.BlockSpec(
            (1, 1), lambda i: (0, 0), memory_space=pltpu.SMEM),
        out_shape=jax.ShapeDtypeStruct((1, 1), jnp.float32),
    )(gathered, center)
    return out[0, 0]


def kernel(center_vecs, pos_idx, neg_idx, output_emb):
    B, EMB = center_vecs.shape
    K = neg_idx.shape[1]
    T = K + 1
    NW = 32
    b_per_w = B // NW
    n_chunks = b_per_w // _CHUNK

    idx_all = jnp.concatenate(
        [pos_idx.astype(jnp.int32)[:, None], neg_idx.astype(jnp.int32)],
        axis=1)
    idx_r = idx_all.reshape(NW, n_chunks, _CHUNK, T).transpose(0, 3, 1, 2)
    idx_r = idx_r.reshape(NW, T * n_chunks, _CHUNK)

    gathered = _sc_gather(output_emb, idx_r, NW, T, n_chunks)
    return _tc_loss(gathered, center_vecs, T, B, EMB, bb=2048)
